# baseline (speedup 1.0000x reference)
import functools

import jax
import jax.numpy as jnp
from jax import lax
from jax.experimental import pallas as pl
from jax.experimental.pallas import tpu as pltpu
from jax.experimental.pallas import tpu_sc as plsc

_NB = 2
_NS = 8


@functools.partial(jax.jit, static_argnames=("hp", "nb"))
def _gather_sc(vp, idx, hp, nb):
    b, h = idx.shape
    dp = vp.shape[1]
    info = plsc.get_sparse_core_info()
    nc = info.num_cores
    nw = nc * info.num_subcores
    rows_per_w = b // nw
    n_chunks = rows_per_w // nb

    mesh = plsc.VectorSubcoreMesh(core_axis_name="c", subcore_axis_name="s")

    @functools.partial(
        pl.kernel,
        mesh=mesh,
        out_type=jax.ShapeDtypeStruct((b, hp, dp), jnp.float32),
        compiler_params=pltpu.CompilerParams(use_tc_tiling_on_sc=False,
                                             needs_layout_passes=False),
        scratch_types=[
            pltpu.VMEM((rows_per_w, h), jnp.int32),
            pltpu.VMEM((_NS, nb, h, dp), jnp.float32),
        ] + [pltpu.SemaphoreType.DMA] * (2 * _NS),
    )
    def k(table_hbm, idx_hbm, out_hbm, idx_all, rows_v, *sems):
        wid = lax.axis_index("s") * nc + lax.axis_index("c")
        base = wid * rows_per_w
        gsem = sems[:_NS]
        wsem = sems[_NS:]

        pltpu.sync_copy(idx_hbm.at[pl.ds(base, rows_per_w)], idx_all)

        def start_gathers(g, slot):
            for j in range(nb):
                pltpu.make_async_copy(
                    table_hbm.at[idx_all.at[g * nb + j]],
                    rows_v.at[slot, j],
                    gsem[slot],
                ).start()

        def wait_gathers(slot):
            pltpu.make_async_copy(
                table_hbm.at[idx_all.at[0]],
                rows_v.at[slot],
                gsem[slot],
            ).wait()

        def start_write(g, slot):
            pltpu.make_async_copy(
                rows_v.at[slot],
                out_hbm.at[pl.ds(base + g * nb, nb), pl.ds(0, h), :],
                wsem[slot],
            ).start()

        def wait_write(slot):
            pltpu.make_async_copy(
                rows_v.at[slot],
                out_hbm.at[pl.ds(base, nb), pl.ds(0, h), :],
                wsem[slot],
            ).wait()

        for u in range(_NS - 1):
            start_gathers(u, u)

        def step(g_dyn, u):
            s3 = (u + _NS - 1) % _NS
            pn = g_dyn + _NS - 1

            @pl.when(jnp.logical_and(pn < n_chunks, g_dyn >= 1))
            def _():
                wait_write(s3)

            @pl.when(pn < n_chunks)
            def _():
                start_gathers(pn, s3)

            wait_gathers(u)
            start_write(g_dyn, u)

        def body(i, carry):
            for u in range(_NS):
                step(_NS * i + u, u)
            return carry

        lax.fori_loop(0, n_chunks // _NS, body, 0)
        for u in range(_NS):
            wait_write(u)

    return k(vp, idx)


def kernel(v, idx):
    b, h = idx.shape
    d = v.shape[1]
    hp = 56
    vp = jnp.pad(v, ((0, 0), (0, 128 - d)))
    y = _gather_sc(vp, idx, hp, _NB)
    return y[:, :h, :d]
